# baseline (device time: 86421 ns/iter reference)
import jax
import jax.numpy as jnp
from jax import lax
from jax.experimental import pallas as pl
from jax.experimental.pallas import tpu as pltpu

N_DEV = 4
EPS = 1e-6


def kernel(partial, gamma):
    _, m, d = partial.shape
    x = partial.reshape(m, d)
    g = gamma.reshape(1, d)
    m_out = m // N_DEV

    def body(x_ref, g_ref, out_ref, comm_ref, send_sems, recv_sems):
        my_x = lax.axis_index("x")
        my_y = lax.axis_index("y")
        my_z = lax.axis_index("z")
        right = lax.rem(my_z + 1, N_DEV)
        left = lax.rem(my_z + N_DEV - 1, N_DEV)

        barrier = pltpu.get_barrier_semaphore()
        for nbr in (left, right):
            pl.semaphore_signal(
                barrier, inc=1,
                device_id=(my_x, my_y, nbr),
                device_id_type=pl.DeviceIdType.MESH,
            )
        pl.semaphore_wait(barrier, 2)

        def chunk(i):
            return x_ref[pl.ds(i * m_out, m_out), :]

        comm_ref[0, :, :] = chunk(left).astype(jnp.bfloat16)

        for s in range(N_DEV - 1):
            rdma = pltpu.make_async_remote_copy(
                src_ref=comm_ref.at[s],
                dst_ref=comm_ref.at[s + 1],
                send_sem=send_sems.at[s],
                recv_sem=recv_sems.at[s + 1],
                device_id=(my_x, my_y, right),
                device_id_type=pl.DeviceIdType.MESH,
            )
            rdma.start()
            rdma.wait()
            rc = lax.rem(my_z + 2 * N_DEV - s - 2, N_DEV)
            if s < N_DEV - 2:
                comm_ref[s + 1, :, :] = (
                    comm_ref[s + 1, :, :] + chunk(rc).astype(jnp.bfloat16)
                )
            else:
                y = comm_ref[s + 1, :, :].astype(jnp.float32) + chunk(rc)
                ms = jnp.mean(y * y, axis=1, keepdims=True)
                out_ref[:, :] = y * lax.rsqrt(ms + EPS) * g_ref[:, :]

    return pl.pallas_call(
        body,
        out_shape=jax.ShapeDtypeStruct((m_out, d), jnp.float32),
        in_specs=[
            pl.BlockSpec(memory_space=pltpu.VMEM),
            pl.BlockSpec(memory_space=pltpu.VMEM),
        ],
        out_specs=pl.BlockSpec(memory_space=pltpu.VMEM),
        scratch_shapes=[
            pltpu.VMEM((N_DEV, m_out, d), jnp.bfloat16),
            pltpu.SemaphoreType.DMA((N_DEV,)),
            pltpu.SemaphoreType.DMA((N_DEV,)),
        ],
        compiler_params=pltpu.CompilerParams(collective_id=0),
    )(x, g)


# device time: 85548 ns/iter; 1.0102x vs baseline; 1.0102x over previous
import jax
import jax.numpy as jnp
from jax import lax
from jax.experimental import pallas as pl
from jax.experimental.pallas import tpu as pltpu

N_DEV = 4
EPS = 1e-6


def kernel(partial, gamma):
    _, m, d = partial.shape
    x = partial.reshape(m, d)
    g = gamma.reshape(1, d)
    m_out = m // N_DEV
    dh = d // 2

    def body(x_ref, g_ref, out_ref, comm_r, comm_l, send_sems, recv_sems):
        my_x = lax.axis_index("x")
        my_y = lax.axis_index("y")
        my_z = lax.axis_index("z")
        right = lax.rem(my_z + 1, N_DEV)
        left = lax.rem(my_z + N_DEV - 1, N_DEV)

        barrier = pltpu.get_barrier_semaphore()
        for nbr in (left, right):
            pl.semaphore_signal(
                barrier, inc=1,
                device_id=(my_x, my_y, nbr),
                device_id_type=pl.DeviceIdType.MESH,
            )
        pl.semaphore_wait(barrier, 2)

        def chunk_lo(i):
            return x_ref[pl.ds(i * m_out, m_out), pl.ds(0, dh)]

        def chunk_hi(i):
            return x_ref[pl.ds(i * m_out, m_out), pl.ds(dh, dh)]

        comm_r[0, :, :] = chunk_lo(left).astype(jnp.bfloat16)
        comm_l[0, :, :] = chunk_hi(right).astype(jnp.bfloat16)

        for s in range(N_DEV - 1):
            rdma_r = pltpu.make_async_remote_copy(
                src_ref=comm_r.at[s],
                dst_ref=comm_r.at[s + 1],
                send_sem=send_sems.at[0, s],
                recv_sem=recv_sems.at[0, s + 1],
                device_id=(my_x, my_y, right),
                device_id_type=pl.DeviceIdType.MESH,
            )
            rdma_l = pltpu.make_async_remote_copy(
                src_ref=comm_l.at[s],
                dst_ref=comm_l.at[s + 1],
                send_sem=send_sems.at[1, s],
                recv_sem=recv_sems.at[1, s + 1],
                device_id=(my_x, my_y, left),
                device_id_type=pl.DeviceIdType.MESH,
            )
            rdma_r.start()
            rdma_l.start()
            rdma_r.wait()
            rdma_l.wait()
            rc_r = lax.rem(my_z + 2 * N_DEV - s - 2, N_DEV)
            rc_l = lax.rem(my_z + s + 2, N_DEV)
            if s < N_DEV - 2:
                comm_r[s + 1, :, :] = (
                    comm_r[s + 1, :, :] + chunk_lo(rc_r).astype(jnp.bfloat16)
                )
                comm_l[s + 1, :, :] = (
                    comm_l[s + 1, :, :] + chunk_hi(rc_l).astype(jnp.bfloat16)
                )
            else:
                y_lo = comm_r[s + 1, :, :].astype(jnp.float32) + chunk_lo(rc_r)
                y_hi = comm_l[s + 1, :, :].astype(jnp.float32) + chunk_hi(rc_l)
                ssq = (
                    jnp.sum(y_lo * y_lo, axis=1, keepdims=True)
                    + jnp.sum(y_hi * y_hi, axis=1, keepdims=True)
                )
                inv = lax.rsqrt(ssq / d + EPS)
                out_ref[:, pl.ds(0, dh)] = y_lo * inv * g_ref[:, pl.ds(0, dh)]
                out_ref[:, pl.ds(dh, dh)] = y_hi * inv * g_ref[:, pl.ds(dh, dh)]

    return pl.pallas_call(
        body,
        out_shape=jax.ShapeDtypeStruct((m_out, d), jnp.float32),
        in_specs=[
            pl.BlockSpec(memory_space=pltpu.VMEM),
            pl.BlockSpec(memory_space=pltpu.VMEM),
        ],
        out_specs=pl.BlockSpec(memory_space=pltpu.VMEM),
        scratch_shapes=[
            pltpu.VMEM((N_DEV, m_out, dh), jnp.bfloat16),
            pltpu.VMEM((N_DEV, m_out, dh), jnp.bfloat16),
            pltpu.SemaphoreType.DMA((2, N_DEV)),
            pltpu.SemaphoreType.DMA((2, N_DEV)),
        ],
        compiler_params=pltpu.CompilerParams(collective_id=0),
    )(x, g)


# device time: 49196 ns/iter; 1.7567x vs baseline; 1.7389x over previous
import jax
import jax.numpy as jnp
from jax import lax
from jax.experimental import pallas as pl
from jax.experimental.pallas import tpu as pltpu

NZ = 4
NR = 8
EPS = 1e-6


def kernel(partial, gamma):
    _, m, d = partial.shape
    x = partial.reshape(m, d)
    g = gamma.reshape(1, d)
    m_out = m // NZ
    dsl = d // NR

    def body(x_ref, g_ref, out_ref,
             p1_stage, p1_recv, p2_recv,
             p1_send_sems, p1_recv_sems, p2_send_sems, p2_recv_sems):
        my_x = lax.axis_index("x")
        my_y = lax.axis_index("y")
        my_z = lax.axis_index("z")
        my_q = my_x * 4 + my_y

        barrier = pltpu.get_barrier_semaphore()
        for o in range(1, NZ):
            zt = lax.rem(my_z + o, NZ)
            pl.semaphore_signal(barrier, inc=1, device_id=(my_x, my_y, zt),
                                device_id_type=pl.DeviceIdType.MESH)
        for o in range(1, NR):
            qt = lax.rem(my_q + o, NR)
            pl.semaphore_signal(barrier, inc=1,
                                device_id=(qt // 4, lax.rem(qt, 4), my_z),
                                device_id_type=pl.DeviceIdType.MESH)
        pl.semaphore_wait(barrier, (NZ - 1) + (NR - 1))

        col0 = my_q * dsl

        p1_sends = []
        for o in range(1, NZ):
            zt = lax.rem(my_z + o, NZ)
            p1_stage[o - 1, :, :] = x_ref[
                pl.ds(zt * m_out, m_out), pl.ds(col0, dsl)
            ].astype(jnp.bfloat16)
            send = pltpu.make_async_remote_copy(
                src_ref=p1_stage.at[o - 1],
                dst_ref=p1_recv.at[NZ - o - 1],
                send_sem=p1_send_sems.at[o - 1],
                recv_sem=p1_recv_sems.at[NZ - o - 1],
                device_id=(my_x, my_y, zt),
                device_id_type=pl.DeviceIdType.MESH,
            )
            send.start()
            p1_sends.append(send)

        for s in range(NZ - 1):
            pltpu.make_async_remote_copy(
                src_ref=p1_stage.at[0],
                dst_ref=p1_recv.at[s],
                send_sem=p1_send_sems.at[0],
                recv_sem=p1_recv_sems.at[s],
                device_id=(my_x, my_y, my_z),
                device_id_type=pl.DeviceIdType.MESH,
            ).wait_recv()

        y_slice = (
            p1_recv[0, :, :].astype(jnp.float32)
            + p1_recv[1, :, :].astype(jnp.float32)
            + p1_recv[2, :, :].astype(jnp.float32)
            + x_ref[pl.ds(my_z * m_out, m_out), pl.ds(col0, dsl)]
        )
        p2_recv[my_q, :, :] = y_slice.astype(jnp.bfloat16)

        p2_sends = []
        for o in range(1, NR):
            qt = lax.rem(my_q + o, NR)
            send = pltpu.make_async_remote_copy(
                src_ref=p2_recv.at[my_q],
                dst_ref=p2_recv.at[my_q],
                send_sem=p2_send_sems.at[o - 1],
                recv_sem=p2_recv_sems.at[my_q],
                device_id=(qt // 4, lax.rem(qt, 4), my_z),
                device_id_type=pl.DeviceIdType.MESH,
            )
            send.start()
            p2_sends.append(send)

        for o in range(1, NR):
            qs = lax.rem(my_q + o, NR)
            pltpu.make_async_remote_copy(
                src_ref=p2_recv.at[my_q],
                dst_ref=p2_recv.at[qs],
                send_sem=p2_send_sems.at[0],
                recv_sem=p2_recv_sems.at[qs],
                device_id=(my_x, my_y, my_z),
                device_id_type=pl.DeviceIdType.MESH,
            ).wait_recv()

        ssq = jnp.zeros((m_out, 1), jnp.float32)
        for j in range(NR):
            yj = p2_recv[j, :, :].astype(jnp.float32)
            ssq = ssq + jnp.sum(yj * yj, axis=1, keepdims=True)
        inv = lax.rsqrt(ssq / d + EPS)
        for j in range(NR):
            out_ref[:, j * dsl:(j + 1) * dsl] = (
                p2_recv[j, :, :].astype(jnp.float32)
                * inv * g_ref[:, j * dsl:(j + 1) * dsl]
            )

        for send in p1_sends + p2_sends:
            send.wait_send()

    return pl.pallas_call(
        body,
        out_shape=jax.ShapeDtypeStruct((m_out, d), jnp.float32),
        in_specs=[
            pl.BlockSpec(memory_space=pltpu.VMEM),
            pl.BlockSpec(memory_space=pltpu.VMEM),
        ],
        out_specs=pl.BlockSpec(memory_space=pltpu.VMEM),
        scratch_shapes=[
            pltpu.VMEM((NZ - 1, m_out, dsl), jnp.bfloat16),
            pltpu.VMEM((NZ - 1, m_out, dsl), jnp.bfloat16),
            pltpu.VMEM((NR, m_out, dsl), jnp.bfloat16),
            pltpu.SemaphoreType.DMA((NZ - 1,)),
            pltpu.SemaphoreType.DMA((NZ - 1,)),
            pltpu.SemaphoreType.DMA((NR - 1,)),
            pltpu.SemaphoreType.DMA((NR,)),
        ],
        compiler_params=pltpu.CompilerParams(collective_id=0),
    )(x, g)
